# Initial kernel scaffold; baseline (speedup 1.0000x reference)
import jax
import jax.numpy as jnp
from jax.experimental import pallas as pl


def kernel(x, wg, w1, w2):
    raise NotImplementedError("write your pallas kernel here")


# speedup vs baseline: 1.3676x; 1.3676x over previous
import functools
import math

import jax
import jax.numpy as jnp
from jax import lax
from jax.experimental import pallas as pl
from jax.experimental.pallas import tpu as pltpu
from jax.experimental.pallas import tpu_sc as plsc

K = 2
CAPACITY_FACTOR = 1.25


def _vgather(vec, idx):
    return lax.gather(
        vec, idx[:, None],
        lax.GatherDimensionNumbers(offset_dims=(), collapsed_slice_dims=(0,),
                                   start_index_map=(0,)),
        (1,), mode=lax.GatherScatterMode.PROMISE_IN_BOUNDS)


def _router_body(caps, x_ref, wg_ref, topi_ref, posp_ref, topw_ref, tot_ref):
    E, CAP, BT = caps
    b = pl.program_id(0)

    @pl.when(b == 0)
    def _init():
        tot_ref[...] = jnp.zeros_like(tot_ref)

    x = x_ref[...]
    wg = wg_ref[...]
    logits = jnp.dot(x, wg, preferred_element_type=jnp.float32)
    m = jnp.max(logits, axis=-1, keepdims=True)
    p = jnp.exp(logits - m)
    gates = p / jnp.sum(p, axis=-1, keepdims=True)

    lane = lax.broadcasted_iota(jnp.int32, gates.shape, 1)
    v1 = jnp.max(gates, axis=-1, keepdims=True)
    i1 = jnp.min(jnp.where(gates == v1, lane, E), axis=-1, keepdims=True)
    g2 = jnp.where(lane == i1, -jnp.inf, gates)
    v2 = jnp.max(g2, axis=-1, keepdims=True)
    i2 = jnp.min(jnp.where(g2 == v2, lane, E), axis=-1, keepdims=True)

    s = v1 + v2
    topw_ref[...] = jnp.concatenate([v1 / s, v2 / s], axis=1)
    topi_ref[...] = jnp.concatenate([i1, i2], axis=1)

    oh1 = (lane == i1).astype(jnp.float32)
    oh2 = (lane == i2).astype(jnp.float32)
    r = lax.broadcasted_iota(jnp.int32, (BT, BT), 0)
    c = lax.broadcasted_iota(jnp.int32, (BT, BT), 1)
    tri = (c < r).astype(jnp.float32)
    prev1 = jnp.dot(tri, oh1, preferred_element_type=jnp.float32)
    prev2 = jnp.dot(tri, oh2, preferred_element_type=jnp.float32)

    carry = tot_ref[...]
    pos1 = jnp.sum(oh1 * (prev1 + carry[0:1, :]), axis=-1, keepdims=True)
    pos2 = jnp.sum(oh2 * (prev2 + carry[1:2, :]), axis=-1, keepdims=True)
    posp_ref[...] = jnp.concatenate([pos1, pos2], axis=1).astype(jnp.int32)

    counts = jnp.concatenate([jnp.sum(oh1, axis=0, keepdims=True),
                              jnp.sum(oh2, axis=0, keepdims=True)], axis=0)
    tot_ref[...] = carry + counts


def _run_router(x, wg, E, CAP, interpret=False):
    T, D = x.shape
    BT = 512
    NB = T // BT
    return pl.pallas_call(
        functools.partial(_router_body, (E, CAP, BT)),
        grid=(NB,),
        in_specs=[
            pl.BlockSpec((BT, D), lambda b: (b, 0)),
            pl.BlockSpec((D, E), lambda b: (0, 0)),
        ],
        out_specs=[
            pl.BlockSpec((BT, K), lambda b: (b, 0)),
            pl.BlockSpec((BT, K), lambda b: (b, 0)),
            pl.BlockSpec((BT, K), lambda b: (b, 0)),
            pl.BlockSpec((K, E), lambda b: (0, 0)),
        ],
        out_shape=[
            jax.ShapeDtypeStruct((T, K), jnp.int32),
            jax.ShapeDtypeStruct((T, K), jnp.int32),
            jax.ShapeDtypeStruct((T, K), jnp.float32),
            jax.ShapeDtypeStruct((K, E), jnp.float32),
        ],
        compiler_params=pltpu.CompilerParams(
            dimension_semantics=("arbitrary",)),
        interpret=interpret,
    )(x, wg)


def _make_dispatch(T, D, E, CAP):
    info = plsc.get_sparse_core_info()
    NW = info.num_cores * info.num_subcores
    CH = 32
    t_per_w = T // NW
    n_chunks = t_per_w // CH
    DUMP = E * CAP
    mesh = plsc.VectorSubcoreMesh(core_axis_name="c", subcore_axis_name="s")

    @functools.partial(
        pl.kernel, mesh=mesh,
        out_type=jax.ShapeDtypeStruct(((E + 1) * CAP, D), jnp.float32),
        scratch_types=[
            pltpu.VMEM((CH, D), jnp.float32),
            pltpu.VMEM((CH,), jnp.int32),
            pltpu.VMEM((CH,), jnp.int32),
            pltpu.VMEM((CH,), jnp.int32),
            pltpu.VMEM((CH,), jnp.int32),
            pltpu.VMEM((CH,), jnp.int32),
            pltpu.VMEM((CH,), jnp.int32),
            pltpu.VMEM((16,), jnp.float32),
            pltpu.SemaphoreType.DMA,
        ],
    )
    def dispatch(x_hbm, ti0_hbm, pp0_hbm, ti1_hbm, pp1_hbm, tot0_hbm,
                 disp_hbm, xb, e0b, p0b, e1b, p1b, s0b, s1b, totv, sem):
        wid = lax.axis_index("s") * info.num_cores + lax.axis_index("c")
        base = wid * t_per_w
        pltpu.sync_copy(tot0_hbm, totv)
        for ch in range(n_chunks):
            tb = base + ch * CH
            pltpu.sync_copy(x_hbm.at[pl.ds(tb, CH)], xb)
            pltpu.sync_copy(ti0_hbm.at[pl.ds(tb, CH)], e0b)
            pltpu.sync_copy(pp0_hbm.at[pl.ds(tb, CH)], p0b)
            pltpu.sync_copy(ti1_hbm.at[pl.ds(tb, CH)], e1b)
            pltpu.sync_copy(pp1_hbm.at[pl.ds(tb, CH)], p1b)
            for h in range(CH // 16):
                sl = pl.ds(h * 16, 16)
                e0 = e0b[sl]
                p0 = p0b[sl]
                s0b[sl] = jnp.where(p0 < CAP, e0 * CAP + p0, DUMP)
                e1 = e1b[sl]
                p1 = p1b[sl] + _vgather(totv[...], e1).astype(jnp.int32)
                s1b[sl] = jnp.where(p1 < CAP, e1 * CAP + p1, DUMP)
            c0 = pltpu.async_copy(xb, disp_hbm.at[s0b], sem)
            c1 = pltpu.async_copy(xb, disp_hbm.at[s1b], sem)
            c0.wait()
            c1.wait()

    return dispatch


def _ffn_body(caps, disp_ref, w1_ref, w2_ref, eo_ref):
    CAP, D, FB = caps
    f = pl.program_id(1)
    dsp = disp_ref[...]
    h = jnp.dot(dsp, w1_ref[0], preferred_element_type=jnp.float32)
    h = jnp.maximum(h, 0.0)
    contrib = jnp.dot(h, w2_ref[0], preferred_element_type=jnp.float32)

    @pl.when(f == 0)
    def _first():
        eo_ref[...] = contrib

    @pl.when(f != 0)
    def _rest():
        eo_ref[...] = eo_ref[...] + contrib


def _run_ffn(disp, w1, w2, E, CAP, interpret=False):
    D = w1.shape[1]
    F = w1.shape[2]
    FB = 512
    NF = F // FB
    return pl.pallas_call(
        functools.partial(_ffn_body, (CAP, D, FB)),
        grid=(E, NF),
        in_specs=[
            pl.BlockSpec((CAP, D), lambda e, f: (e, 0)),
            pl.BlockSpec((1, D, FB), lambda e, f: (e, 0, f)),
            pl.BlockSpec((1, FB, D), lambda e, f: (e, f, 0)),
        ],
        out_specs=pl.BlockSpec((CAP, D), lambda e, f: (e, 0)),
        out_shape=jax.ShapeDtypeStruct((E * CAP, D), jnp.float32),
        compiler_params=pltpu.CompilerParams(
            dimension_semantics=("parallel", "arbitrary")),
        interpret=interpret,
    )(disp, w1, w2)


def _make_combine(T, D, E, CAP):
    info = plsc.get_sparse_core_info()
    NW = info.num_cores * info.num_subcores
    CH = 32
    t_per_w = T // NW
    n_chunks = t_per_w // CH
    mesh = plsc.VectorSubcoreMesh(core_axis_name="c", subcore_axis_name="s")

    @functools.partial(
        pl.kernel, mesh=mesh,
        out_type=jax.ShapeDtypeStruct((T, D), jnp.float32),
        scratch_types=[
            pltpu.VMEM((CH, D), jnp.float32),
            pltpu.VMEM((CH, D), jnp.float32),
            pltpu.VMEM((CH,), jnp.int32),
            pltpu.VMEM((CH,), jnp.int32),
            pltpu.VMEM((CH,), jnp.int32),
            pltpu.VMEM((CH,), jnp.int32),
            pltpu.VMEM((CH,), jnp.float32),
            pltpu.VMEM((CH,), jnp.float32),
            pltpu.VMEM((CH,), jnp.int32),
            pltpu.VMEM((CH,), jnp.int32),
            pltpu.VMEM((CH,), jnp.float32),
            pltpu.VMEM((CH,), jnp.float32),
            pltpu.VMEM((16,), jnp.float32),
            pltpu.SemaphoreType.DMA,
        ],
    )
    def combine(eo_hbm, ti0_hbm, pp0_hbm, ti1_hbm, pp1_hbm, wr0_hbm, wr1_hbm,
                tot0_hbm, y_hbm, g0, g1, e0b, p0b, e1b, p1b, w0r, w1r,
                r0b, r1b, w0b, w1b, totv, sem):
        wid = lax.axis_index("s") * info.num_cores + lax.axis_index("c")
        base = wid * t_per_w
        pltpu.sync_copy(tot0_hbm, totv)
        for ch in range(n_chunks):
            tb = base + ch * CH
            pltpu.sync_copy(ti0_hbm.at[pl.ds(tb, CH)], e0b)
            pltpu.sync_copy(pp0_hbm.at[pl.ds(tb, CH)], p0b)
            pltpu.sync_copy(ti1_hbm.at[pl.ds(tb, CH)], e1b)
            pltpu.sync_copy(pp1_hbm.at[pl.ds(tb, CH)], p1b)
            pltpu.sync_copy(wr0_hbm.at[pl.ds(tb, CH)], w0r)
            pltpu.sync_copy(wr1_hbm.at[pl.ds(tb, CH)], w1r)
            for h in range(CH // 16):
                sl = pl.ds(h * 16, 16)
                e0 = e0b[sl]
                p0 = p0b[sl]
                r0b[sl] = e0 * CAP + jnp.minimum(p0, CAP - 1)
                w0b[sl] = jnp.where(p0 < CAP, w0r[sl], 0.0)
                e1 = e1b[sl]
                p1 = p1b[sl] + _vgather(totv[...], e1).astype(jnp.int32)
                r1b[sl] = e1 * CAP + jnp.minimum(p1, CAP - 1)
                w1b[sl] = jnp.where(p1 < CAP, w1r[sl], 0.0)
            c0 = pltpu.async_copy(eo_hbm.at[r0b], g0, sem)
            c1 = pltpu.async_copy(eo_hbm.at[r1b], g1, sem)
            c0.wait()
            c1.wait()

            for h in range(CH // 16):
                wv0 = w0b[pl.ds(h * 16, 16)]
                wv1 = w1b[pl.ds(h * 16, 16)]

                def row(r, _, wv0=wv0, wv1=wv1, h=h):
                    i = h * 16 + r
                    rv = jnp.full((16,), r, dtype=jnp.int32)
                    w0v = _vgather(wv0, rv)
                    w1v = _vgather(wv1, rv)
                    for j in range(D // 16):
                        sl = pl.ds(j * 16, 16)
                        g0[i, sl] = g0[i, sl] * w0v + g1[i, sl] * w1v
                    return 0

                lax.fori_loop(0, 16, row, 0)
            pltpu.sync_copy(g0, y_hbm.at[pl.ds(tb, CH)])

    return combine


def kernel(x, wg, w1, w2):
    T, D = x.shape
    E = wg.shape[1]
    CAP = int(math.ceil(T * K / E * CAPACITY_FACTOR))

    topi, posp, topw, totals = _run_router(x, wg, E, CAP)
    ti0 = topi[:, 0]
    ti1 = topi[:, 1]
    pp0 = posp[:, 0]
    pp1 = posp[:, 1]
    wr0 = topw[:, 0]
    wr1 = topw[:, 1]
    tot0 = totals[0]

    disp = _make_dispatch(T, D, E, CAP)(x, ti0, pp0, ti1, pp1, tot0)
    eo = _run_ffn(disp, w1, w2, E, CAP)
    y = _make_combine(T, D, E, CAP)(eo, ti0, pp0, ti1, pp1, wr0, wr1, tot0)
    return y


# speedup vs baseline: 1.4814x; 1.0832x over previous
import functools
import math

import jax
import jax.numpy as jnp
from jax import lax
from jax.experimental import pallas as pl
from jax.experimental.pallas import tpu as pltpu
from jax.experimental.pallas import tpu_sc as plsc

K = 2
CAPACITY_FACTOR = 1.25


def _vgather(vec, idx):
    return lax.gather(
        vec, idx[:, None],
        lax.GatherDimensionNumbers(offset_dims=(), collapsed_slice_dims=(0,),
                                   start_index_map=(0,)),
        (1,), mode=lax.GatherScatterMode.PROMISE_IN_BOUNDS)


def _router_body(caps, x_ref, wg_ref, topi_ref, posp_ref, topw_ref, tot_ref):
    E, CAP, BT = caps
    b = pl.program_id(0)

    @pl.when(b == 0)
    def _init():
        tot_ref[...] = jnp.zeros_like(tot_ref)

    x = x_ref[...]
    wg = wg_ref[...]
    logits = jnp.dot(x, wg, preferred_element_type=jnp.float32)
    m = jnp.max(logits, axis=-1, keepdims=True)
    p = jnp.exp(logits - m)
    gates = p / jnp.sum(p, axis=-1, keepdims=True)

    lane = lax.broadcasted_iota(jnp.int32, gates.shape, 1)
    v1 = jnp.max(gates, axis=-1, keepdims=True)
    i1 = jnp.min(jnp.where(gates == v1, lane, E), axis=-1, keepdims=True)
    g2 = jnp.where(lane == i1, -jnp.inf, gates)
    v2 = jnp.max(g2, axis=-1, keepdims=True)
    i2 = jnp.min(jnp.where(g2 == v2, lane, E), axis=-1, keepdims=True)

    s = v1 + v2
    topw_ref[...] = jnp.concatenate([v1 / s, v2 / s], axis=1)
    topi_ref[...] = jnp.concatenate([i1, i2], axis=1)

    oh1 = (lane == i1).astype(jnp.float32)
    oh2 = (lane == i2).astype(jnp.float32)
    r = lax.broadcasted_iota(jnp.int32, (BT, BT), 0)
    c = lax.broadcasted_iota(jnp.int32, (BT, BT), 1)
    tri = (c < r).astype(jnp.float32)
    prev1 = jnp.dot(tri, oh1, preferred_element_type=jnp.float32)
    prev2 = jnp.dot(tri, oh2, preferred_element_type=jnp.float32)

    carry = tot_ref[...]
    pos1 = jnp.sum(oh1 * (prev1 + carry[0:1, :]), axis=-1, keepdims=True)
    pos2 = jnp.sum(oh2 * (prev2 + carry[1:2, :]), axis=-1, keepdims=True)
    posp_ref[...] = jnp.concatenate([pos1, pos2], axis=1).astype(jnp.int32)

    counts = jnp.concatenate([jnp.sum(oh1, axis=0, keepdims=True),
                              jnp.sum(oh2, axis=0, keepdims=True)], axis=0)
    tot_ref[...] = carry + counts


def _run_router(x, wg, E, CAP, interpret=False):
    T, D = x.shape
    BT = 512
    NB = T // BT
    return pl.pallas_call(
        functools.partial(_router_body, (E, CAP, BT)),
        grid=(NB,),
        in_specs=[
            pl.BlockSpec((BT, D), lambda b: (b, 0)),
            pl.BlockSpec((D, E), lambda b: (0, 0)),
        ],
        out_specs=[
            pl.BlockSpec((BT, K), lambda b: (b, 0)),
            pl.BlockSpec((BT, K), lambda b: (b, 0)),
            pl.BlockSpec((BT, K), lambda b: (b, 0)),
            pl.BlockSpec((K, E), lambda b: (0, 0)),
        ],
        out_shape=[
            jax.ShapeDtypeStruct((T, K), jnp.int32),
            jax.ShapeDtypeStruct((T, K), jnp.int32),
            jax.ShapeDtypeStruct((T, K), jnp.float32),
            jax.ShapeDtypeStruct((K, E), jnp.float32),
        ],
        compiler_params=pltpu.CompilerParams(
            dimension_semantics=("arbitrary",)),
        interpret=interpret,
    )(x, wg)


def _make_dispatch(T, D, E, CAP):
    info = plsc.get_sparse_core_info()
    NW = info.num_cores * info.num_subcores
    CH = 32
    t_per_w = T // NW
    n_chunks = t_per_w // CH
    DUMP = E * CAP
    mesh = plsc.VectorSubcoreMesh(core_axis_name="c", subcore_axis_name="s")

    @functools.partial(
        pl.kernel, mesh=mesh,
        out_type=jax.ShapeDtypeStruct(((E + 1) * CAP, D), jnp.float32),
        scratch_types=[
            pltpu.VMEM((CH, D), jnp.float32),
            pltpu.VMEM((CH,), jnp.int32),
            pltpu.VMEM((CH,), jnp.int32),
            pltpu.VMEM((CH,), jnp.int32),
            pltpu.VMEM((CH,), jnp.int32),
            pltpu.VMEM((CH,), jnp.int32),
            pltpu.VMEM((CH,), jnp.int32),
            pltpu.VMEM((16,), jnp.float32),
            pltpu.SemaphoreType.DMA,
        ],
    )
    def dispatch(x_hbm, ti0_hbm, pp0_hbm, ti1_hbm, pp1_hbm, tot0_hbm,
                 disp_hbm, xb, e0b, p0b, e1b, p1b, s0b, s1b, totv, sem):
        wid = lax.axis_index("s") * info.num_cores + lax.axis_index("c")
        base = wid * t_per_w
        pltpu.sync_copy(tot0_hbm, totv)
        for ch in range(n_chunks):
            tb = base + ch * CH
            pltpu.sync_copy(x_hbm.at[pl.ds(tb, CH)], xb)
            pltpu.sync_copy(ti0_hbm.at[pl.ds(tb, CH)], e0b)
            pltpu.sync_copy(pp0_hbm.at[pl.ds(tb, CH)], p0b)
            pltpu.sync_copy(ti1_hbm.at[pl.ds(tb, CH)], e1b)
            pltpu.sync_copy(pp1_hbm.at[pl.ds(tb, CH)], p1b)
            for h in range(CH // 16):
                sl = pl.ds(h * 16, 16)
                e0 = e0b[sl]
                p0 = p0b[sl]
                s0b[sl] = jnp.where(p0 < CAP, e0 * CAP + p0, DUMP)
                e1 = e1b[sl]
                p1 = p1b[sl] + _vgather(totv[...], e1).astype(jnp.int32)
                s1b[sl] = jnp.where(p1 < CAP, e1 * CAP + p1, DUMP)
            c0 = pltpu.async_copy(xb, disp_hbm.at[s0b], sem)
            c1 = pltpu.async_copy(xb, disp_hbm.at[s1b], sem)
            c0.wait()
            c1.wait()

    return dispatch


def _ffn_body(caps, disp_ref, w1_ref, w2_ref, eo_ref):
    CAP, D, FB = caps
    f = pl.program_id(1)
    dsp = disp_ref[...]
    h = jnp.dot(dsp, w1_ref[0], preferred_element_type=jnp.float32)
    h = jnp.maximum(h, 0.0)
    contrib = jnp.dot(h, w2_ref[0], preferred_element_type=jnp.float32)

    @pl.when(f == 0)
    def _first():
        eo_ref[...] = contrib

    @pl.when(f != 0)
    def _rest():
        eo_ref[...] = eo_ref[...] + contrib


def _run_ffn(disp, w1, w2, E, CAP, interpret=False):
    D = w1.shape[1]
    F = w1.shape[2]
    FB = 1024
    NF = F // FB
    return pl.pallas_call(
        functools.partial(_ffn_body, (CAP, D, FB)),
        grid=(E, NF),
        in_specs=[
            pl.BlockSpec((CAP, D), lambda e, f: (e, 0)),
            pl.BlockSpec((1, D, FB), lambda e, f: (e, 0, f)),
            pl.BlockSpec((1, FB, D), lambda e, f: (e, f, 0)),
        ],
        out_specs=pl.BlockSpec((CAP, D), lambda e, f: (e, 0)),
        out_shape=jax.ShapeDtypeStruct((E * CAP, D), jnp.float32),
        compiler_params=pltpu.CompilerParams(
            dimension_semantics=("parallel", "arbitrary")),
        interpret=interpret,
    )(disp, w1, w2)


def _make_combine(T, D, E, CAP):
    info = plsc.get_sparse_core_info()
    NW = info.num_cores * info.num_subcores
    CH = 32
    t_per_w = T // NW
    n_chunks = t_per_w // CH
    mesh = plsc.VectorSubcoreMesh(core_axis_name="c", subcore_axis_name="s")

    @functools.partial(
        pl.kernel, mesh=mesh,
        out_type=jax.ShapeDtypeStruct((T, D), jnp.float32),
        scratch_types=[
            pltpu.VMEM((CH, D), jnp.float32),
            pltpu.VMEM((CH, D), jnp.float32),
            pltpu.VMEM((CH,), jnp.int32),
            pltpu.VMEM((CH,), jnp.int32),
            pltpu.VMEM((CH,), jnp.int32),
            pltpu.VMEM((CH,), jnp.int32),
            pltpu.VMEM((CH,), jnp.float32),
            pltpu.VMEM((CH,), jnp.float32),
            pltpu.VMEM((CH,), jnp.int32),
            pltpu.VMEM((CH,), jnp.int32),
            pltpu.VMEM((CH,), jnp.float32),
            pltpu.VMEM((CH,), jnp.float32),
            pltpu.VMEM((16,), jnp.float32),
            pltpu.SemaphoreType.DMA,
        ],
    )
    def combine(eo_hbm, ti0_hbm, pp0_hbm, ti1_hbm, pp1_hbm, wr0_hbm, wr1_hbm,
                tot0_hbm, y_hbm, g0, g1, e0b, p0b, e1b, p1b, w0r, w1r,
                r0b, r1b, w0b, w1b, totv, sem):
        wid = lax.axis_index("s") * info.num_cores + lax.axis_index("c")
        base = wid * t_per_w
        pltpu.sync_copy(tot0_hbm, totv)
        for ch in range(n_chunks):
            tb = base + ch * CH
            pltpu.sync_copy(ti0_hbm.at[pl.ds(tb, CH)], e0b)
            pltpu.sync_copy(pp0_hbm.at[pl.ds(tb, CH)], p0b)
            pltpu.sync_copy(ti1_hbm.at[pl.ds(tb, CH)], e1b)
            pltpu.sync_copy(pp1_hbm.at[pl.ds(tb, CH)], p1b)
            pltpu.sync_copy(wr0_hbm.at[pl.ds(tb, CH)], w0r)
            pltpu.sync_copy(wr1_hbm.at[pl.ds(tb, CH)], w1r)
            for h in range(CH // 16):
                sl = pl.ds(h * 16, 16)
                e0 = e0b[sl]
                p0 = p0b[sl]
                r0b[sl] = e0 * CAP + jnp.minimum(p0, CAP - 1)
                w0b[sl] = jnp.where(p0 < CAP, w0r[sl], 0.0)
                e1 = e1b[sl]
                p1 = p1b[sl] + _vgather(totv[...], e1).astype(jnp.int32)
                r1b[sl] = e1 * CAP + jnp.minimum(p1, CAP - 1)
                w1b[sl] = jnp.where(p1 < CAP, w1r[sl], 0.0)
            c0 = pltpu.async_copy(eo_hbm.at[r0b], g0, sem)
            c1 = pltpu.async_copy(eo_hbm.at[r1b], g1, sem)
            c0.wait()
            c1.wait()

            for h in range(CH // 16):
                wv0 = w0b[pl.ds(h * 16, 16)]
                wv1 = w1b[pl.ds(h * 16, 16)]

                def row(r, _, wv0=wv0, wv1=wv1, h=h):
                    i = h * 16 + r
                    rv = jnp.full((16,), r, dtype=jnp.int32)
                    w0v = _vgather(wv0, rv)
                    w1v = _vgather(wv1, rv)
                    for j in range(D // 16):
                        sl = pl.ds(j * 16, 16)
                        g0[i, sl] = g0[i, sl] * w0v + g1[i, sl] * w1v
                    return 0

                lax.fori_loop(0, 16, row, 0)
            pltpu.sync_copy(g0, y_hbm.at[pl.ds(tb, CH)])

    return combine


def kernel(x, wg, w1, w2):
    T, D = x.shape
    E = wg.shape[1]
    CAP = int(math.ceil(T * K / E * CAPACITY_FACTOR))

    topi, posp, topw, totals = _run_router(x, wg, E, CAP)
    ti0 = topi[:, 0]
    ti1 = topi[:, 1]
    pp0 = posp[:, 0]
    pp1 = posp[:, 1]
    wr0 = topw[:, 0]
    wr1 = topw[:, 1]
    tot0 = totals[0]

    disp = _make_dispatch(T, D, E, CAP)(x, ti0, pp0, ti1, pp1, tot0)
    eo = _run_ffn(disp, w1, w2, E, CAP)
    y = _make_combine(T, D, E, CAP)(eo, ti0, pp0, ti1, pp1, wr0, wr1, tot0)
    return y


# speedup vs baseline: 1.5578x; 1.0516x over previous
import functools
import math

import jax
import jax.numpy as jnp
from jax import lax
from jax.experimental import pallas as pl
from jax.experimental.pallas import tpu as pltpu
from jax.experimental.pallas import tpu_sc as plsc

K = 2
CAPACITY_FACTOR = 1.25


def _vgather(vec, idx):
    return lax.gather(
        vec, idx[:, None],
        lax.GatherDimensionNumbers(offset_dims=(), collapsed_slice_dims=(0,),
                                   start_index_map=(0,)),
        (1,), mode=lax.GatherScatterMode.PROMISE_IN_BOUNDS)


def _router_body(caps, x_ref, wg_ref, topi_ref, posp_ref, topw_ref, tot_ref):
    E, CAP, BT = caps
    b = pl.program_id(0)

    @pl.when(b == 0)
    def _init():
        tot_ref[...] = jnp.zeros_like(tot_ref)

    x = x_ref[...]
    wg = wg_ref[...]
    logits = jnp.dot(x, wg, preferred_element_type=jnp.float32)
    m = jnp.max(logits, axis=-1, keepdims=True)
    p = jnp.exp(logits - m)
    gates = p / jnp.sum(p, axis=-1, keepdims=True)

    lane = lax.broadcasted_iota(jnp.int32, gates.shape, 1)
    v1 = jnp.max(gates, axis=-1, keepdims=True)
    i1 = jnp.min(jnp.where(gates == v1, lane, E), axis=-1, keepdims=True)
    g2 = jnp.where(lane == i1, -jnp.inf, gates)
    v2 = jnp.max(g2, axis=-1, keepdims=True)
    i2 = jnp.min(jnp.where(g2 == v2, lane, E), axis=-1, keepdims=True)

    s = v1 + v2
    topw_ref[...] = jnp.concatenate([v1 / s, v2 / s], axis=1)
    topi_ref[...] = jnp.concatenate([i1, i2], axis=1)

    oh1 = (lane == i1).astype(jnp.float32)
    oh2 = (lane == i2).astype(jnp.float32)
    r = lax.broadcasted_iota(jnp.int32, (BT, BT), 0)
    c = lax.broadcasted_iota(jnp.int32, (BT, BT), 1)
    tri = (c < r).astype(jnp.float32)
    prev1 = jnp.dot(tri, oh1, preferred_element_type=jnp.float32)
    prev2 = jnp.dot(tri, oh2, preferred_element_type=jnp.float32)

    carry = tot_ref[...]
    pos1 = jnp.sum(oh1 * (prev1 + carry[0:1, :]), axis=-1, keepdims=True)
    pos2 = jnp.sum(oh2 * (prev2 + carry[1:2, :]), axis=-1, keepdims=True)
    posp_ref[...] = jnp.concatenate([pos1, pos2], axis=1).astype(jnp.int32)

    counts = jnp.concatenate([jnp.sum(oh1, axis=0, keepdims=True),
                              jnp.sum(oh2, axis=0, keepdims=True)], axis=0)
    tot_ref[...] = carry + counts


def _run_router(x, wg, E, CAP, interpret=False):
    T, D = x.shape
    BT = 512
    NB = T // BT
    return pl.pallas_call(
        functools.partial(_router_body, (E, CAP, BT)),
        grid=(NB,),
        in_specs=[
            pl.BlockSpec((BT, D), lambda b: (b, 0)),
            pl.BlockSpec((D, E), lambda b: (0, 0)),
        ],
        out_specs=[
            pl.BlockSpec((BT, K), lambda b: (b, 0)),
            pl.BlockSpec((BT, K), lambda b: (b, 0)),
            pl.BlockSpec((BT, K), lambda b: (b, 0)),
            pl.BlockSpec((K, E), lambda b: (0, 0)),
        ],
        out_shape=[
            jax.ShapeDtypeStruct((T, K), jnp.int32),
            jax.ShapeDtypeStruct((T, K), jnp.int32),
            jax.ShapeDtypeStruct((T, K), jnp.float32),
            jax.ShapeDtypeStruct((K, E), jnp.float32),
        ],
        compiler_params=pltpu.CompilerParams(
            dimension_semantics=("arbitrary",)),
        interpret=interpret,
    )(x, wg)


def _make_dispatch(T, D, E, CAP):
    info = plsc.get_sparse_core_info()
    NW = info.num_cores * info.num_subcores
    CH = 32
    t_per_w = T // NW
    n_chunks = t_per_w // CH
    DUMP = E * CAP
    mesh = plsc.VectorSubcoreMesh(core_axis_name="c", subcore_axis_name="s")

    @functools.partial(
        pl.kernel, mesh=mesh,
        out_type=jax.ShapeDtypeStruct(((E + 1) * CAP, D), jnp.float32),
        scratch_types=[
            pltpu.VMEM((CH, D), jnp.float32),
            pltpu.VMEM((CH,), jnp.int32),
            pltpu.VMEM((CH,), jnp.int32),
            pltpu.VMEM((CH,), jnp.int32),
            pltpu.VMEM((CH,), jnp.int32),
            pltpu.VMEM((CH,), jnp.int32),
            pltpu.VMEM((CH,), jnp.int32),
            pltpu.VMEM((16,), jnp.float32),
            pltpu.SemaphoreType.DMA,
        ],
    )
    def dispatch(x_hbm, ti0_hbm, pp0_hbm, ti1_hbm, pp1_hbm, tot0_hbm,
                 disp_hbm, xb, e0b, p0b, e1b, p1b, s0b, s1b, totv, sem):
        wid = lax.axis_index("s") * info.num_cores + lax.axis_index("c")
        base = wid * t_per_w
        pltpu.sync_copy(tot0_hbm, totv)
        for ch in range(n_chunks):
            tb = base + ch * CH
            pltpu.sync_copy(x_hbm.at[pl.ds(tb, CH)], xb)
            pltpu.sync_copy(ti0_hbm.at[pl.ds(tb, CH)], e0b)
            pltpu.sync_copy(pp0_hbm.at[pl.ds(tb, CH)], p0b)
            pltpu.sync_copy(ti1_hbm.at[pl.ds(tb, CH)], e1b)
            pltpu.sync_copy(pp1_hbm.at[pl.ds(tb, CH)], p1b)
            for h in range(CH // 16):
                sl = pl.ds(h * 16, 16)
                e0 = e0b[sl]
                p0 = p0b[sl]
                s0b[sl] = jnp.where(p0 < CAP, e0 * CAP + p0, DUMP)
                e1 = e1b[sl]
                p1 = p1b[sl] + _vgather(totv[...], e1).astype(jnp.int32)
                s1b[sl] = jnp.where(p1 < CAP, e1 * CAP + p1, DUMP)
            c0 = pltpu.async_copy(xb, disp_hbm.at[s0b], sem)
            c1 = pltpu.async_copy(xb, disp_hbm.at[s1b], sem)
            c0.wait()
            c1.wait()

    return dispatch


def _ffn_body(caps, disp_ref, w1_ref, w2_ref, eo_ref):
    CAP, D, FB = caps
    f = pl.program_id(1)
    dsp = disp_ref[...]
    h = jnp.dot(dsp, w1_ref[0], preferred_element_type=jnp.float32)
    h = jnp.maximum(h, 0.0)
    contrib = jnp.dot(h, w2_ref[0], preferred_element_type=jnp.float32)

    @pl.when(f == 0)
    def _first():
        eo_ref[...] = contrib

    @pl.when(f != 0)
    def _rest():
        eo_ref[...] = eo_ref[...] + contrib


def _run_ffn(disp, w1, w2, E, CAP, interpret=False):
    D = w1.shape[1]
    F = w1.shape[2]
    FB = 2048
    NF = F // FB
    return pl.pallas_call(
        functools.partial(_ffn_body, (CAP, D, FB)),
        grid=(E, NF),
        in_specs=[
            pl.BlockSpec((CAP, D), lambda e, f: (e, 0)),
            pl.BlockSpec((1, D, FB), lambda e, f: (e, 0, f)),
            pl.BlockSpec((1, FB, D), lambda e, f: (e, f, 0)),
        ],
        out_specs=pl.BlockSpec((CAP, D), lambda e, f: (e, 0)),
        out_shape=jax.ShapeDtypeStruct((E * CAP, D), jnp.float32),
        compiler_params=pltpu.CompilerParams(
            dimension_semantics=("parallel", "arbitrary")),
        interpret=interpret,
    )(disp, w1, w2)


def _make_combine(T, D, E, CAP):
    info = plsc.get_sparse_core_info()
    NW = info.num_cores * info.num_subcores
    CH = 32
    t_per_w = T // NW
    n_chunks = t_per_w // CH
    mesh = plsc.VectorSubcoreMesh(core_axis_name="c", subcore_axis_name="s")

    @functools.partial(
        pl.kernel, mesh=mesh,
        out_type=jax.ShapeDtypeStruct((T, D), jnp.float32),
        scratch_types=[
            pltpu.VMEM((CH, D), jnp.float32),
            pltpu.VMEM((CH, D), jnp.float32),
            pltpu.VMEM((CH,), jnp.int32),
            pltpu.VMEM((CH,), jnp.int32),
            pltpu.VMEM((CH,), jnp.int32),
            pltpu.VMEM((CH,), jnp.int32),
            pltpu.VMEM((CH,), jnp.float32),
            pltpu.VMEM((CH,), jnp.float32),
            pltpu.VMEM((CH,), jnp.int32),
            pltpu.VMEM((CH,), jnp.int32),
            pltpu.VMEM((CH,), jnp.float32),
            pltpu.VMEM((CH,), jnp.float32),
            pltpu.VMEM((16,), jnp.float32),
            pltpu.SemaphoreType.DMA,
        ],
    )
    def combine(eo_hbm, ti0_hbm, pp0_hbm, ti1_hbm, pp1_hbm, wr0_hbm, wr1_hbm,
                tot0_hbm, y_hbm, g0, g1, e0b, p0b, e1b, p1b, w0r, w1r,
                r0b, r1b, w0b, w1b, totv, sem):
        wid = lax.axis_index("s") * info.num_cores + lax.axis_index("c")
        base = wid * t_per_w
        pltpu.sync_copy(tot0_hbm, totv)
        for ch in range(n_chunks):
            tb = base + ch * CH
            pltpu.sync_copy(ti0_hbm.at[pl.ds(tb, CH)], e0b)
            pltpu.sync_copy(pp0_hbm.at[pl.ds(tb, CH)], p0b)
            pltpu.sync_copy(ti1_hbm.at[pl.ds(tb, CH)], e1b)
            pltpu.sync_copy(pp1_hbm.at[pl.ds(tb, CH)], p1b)
            pltpu.sync_copy(wr0_hbm.at[pl.ds(tb, CH)], w0r)
            pltpu.sync_copy(wr1_hbm.at[pl.ds(tb, CH)], w1r)
            for h in range(CH // 16):
                sl = pl.ds(h * 16, 16)
                e0 = e0b[sl]
                p0 = p0b[sl]
                r0b[sl] = e0 * CAP + jnp.minimum(p0, CAP - 1)
                w0b[sl] = jnp.where(p0 < CAP, w0r[sl], 0.0)
                e1 = e1b[sl]
                p1 = p1b[sl] + _vgather(totv[...], e1).astype(jnp.int32)
                r1b[sl] = e1 * CAP + jnp.minimum(p1, CAP - 1)
                w1b[sl] = jnp.where(p1 < CAP, w1r[sl], 0.0)
            c0 = pltpu.async_copy(eo_hbm.at[r0b], g0, sem)
            c1 = pltpu.async_copy(eo_hbm.at[r1b], g1, sem)
            c0.wait()
            c1.wait()

            for h in range(CH // 16):
                wv0 = w0b[pl.ds(h * 16, 16)]
                wv1 = w1b[pl.ds(h * 16, 16)]

                def row(r, _, wv0=wv0, wv1=wv1, h=h):
                    i = h * 16 + r
                    rv = jnp.full((16,), r, dtype=jnp.int32)
                    w0v = _vgather(wv0, rv)
                    w1v = _vgather(wv1, rv)
                    for j in range(D // 16):
                        sl = pl.ds(j * 16, 16)
                        g0[i, sl] = g0[i, sl] * w0v + g1[i, sl] * w1v
                    return 0

                lax.fori_loop(0, 16, row, 0)
            pltpu.sync_copy(g0, y_hbm.at[pl.ds(tb, CH)])

    return combine


def kernel(x, wg, w1, w2):
    T, D = x.shape
    E = wg.shape[1]
    CAP = int(math.ceil(T * K / E * CAPACITY_FACTOR))

    topi, posp, topw, totals = _run_router(x, wg, E, CAP)
    ti0 = topi[:, 0]
    ti1 = topi[:, 1]
    pp0 = posp[:, 0]
    pp1 = posp[:, 1]
    wr0 = topw[:, 0]
    wr1 = topw[:, 1]
    tot0 = totals[0]

    disp = _make_dispatch(T, D, E, CAP)(x, ti0, pp0, ti1, pp1, tot0)
    eo = _run_ffn(disp, w1, w2, E, CAP)
    y = _make_combine(T, D, E, CAP)(eo, ti0, pp0, ti1, pp1, wr0, wr1, tot0)
    return y


# speedup vs baseline: 1.7010x; 1.0919x over previous
import functools
import math

import jax
import jax.numpy as jnp
from jax import lax
from jax.experimental import pallas as pl
from jax.experimental.pallas import tpu as pltpu
from jax.experimental.pallas import tpu_sc as plsc

K = 2
CAPACITY_FACTOR = 1.25


def _vgather(vec, idx):
    return lax.gather(
        vec, idx[:, None],
        lax.GatherDimensionNumbers(offset_dims=(), collapsed_slice_dims=(0,),
                                   start_index_map=(0,)),
        (1,), mode=lax.GatherScatterMode.PROMISE_IN_BOUNDS)


def _router_body(caps, x_ref, wg_ref, topi_ref, posp_ref, topw_ref, tot_ref):
    E, CAP, BT = caps
    b = pl.program_id(0)

    @pl.when(b == 0)
    def _init():
        tot_ref[...] = jnp.zeros_like(tot_ref)

    x = x_ref[...]
    wg = wg_ref[...]
    logits = jnp.dot(x, wg, preferred_element_type=jnp.float32)
    m = jnp.max(logits, axis=-1, keepdims=True)
    p = jnp.exp(logits - m)
    gates = p / jnp.sum(p, axis=-1, keepdims=True)

    lane = lax.broadcasted_iota(jnp.int32, gates.shape, 1)
    v1 = jnp.max(gates, axis=-1, keepdims=True)
    i1 = jnp.min(jnp.where(gates == v1, lane, E), axis=-1, keepdims=True)
    g2 = jnp.where(lane == i1, -jnp.inf, gates)
    v2 = jnp.max(g2, axis=-1, keepdims=True)
    i2 = jnp.min(jnp.where(g2 == v2, lane, E), axis=-1, keepdims=True)

    s = v1 + v2
    topw_ref[...] = jnp.concatenate([v1 / s, v2 / s], axis=1)
    topi_ref[...] = jnp.concatenate([i1, i2], axis=1)

    oh1 = (lane == i1).astype(jnp.float32)
    oh2 = (lane == i2).astype(jnp.float32)
    r = lax.broadcasted_iota(jnp.int32, (BT, BT), 0)
    c = lax.broadcasted_iota(jnp.int32, (BT, BT), 1)
    tri = (c < r).astype(jnp.float32)
    prev1 = jnp.dot(tri, oh1, preferred_element_type=jnp.float32)
    prev2 = jnp.dot(tri, oh2, preferred_element_type=jnp.float32)

    carry = tot_ref[...]
    pos1 = jnp.sum(oh1 * (prev1 + carry[0:1, :]), axis=-1, keepdims=True)
    pos2 = jnp.sum(oh2 * (prev2 + carry[1:2, :]), axis=-1, keepdims=True)
    posp_ref[...] = jnp.concatenate([pos1, pos2], axis=1).astype(jnp.int32)

    counts = jnp.concatenate([jnp.sum(oh1, axis=0, keepdims=True),
                              jnp.sum(oh2, axis=0, keepdims=True)], axis=0)
    tot_ref[...] = carry + counts


def _run_router(x, wg, E, CAP, interpret=False):
    T, D = x.shape
    BT = 512
    NB = T // BT
    return pl.pallas_call(
        functools.partial(_router_body, (E, CAP, BT)),
        grid=(NB,),
        in_specs=[
            pl.BlockSpec((BT, D), lambda b: (b, 0)),
            pl.BlockSpec((D, E), lambda b: (0, 0)),
        ],
        out_specs=[
            pl.BlockSpec((BT, K), lambda b: (b, 0)),
            pl.BlockSpec((BT, K), lambda b: (b, 0)),
            pl.BlockSpec((BT, K), lambda b: (b, 0)),
            pl.BlockSpec((K, E), lambda b: (0, 0)),
        ],
        out_shape=[
            jax.ShapeDtypeStruct((T, K), jnp.int32),
            jax.ShapeDtypeStruct((T, K), jnp.int32),
            jax.ShapeDtypeStruct((T, K), jnp.float32),
            jax.ShapeDtypeStruct((K, E), jnp.float32),
        ],
        compiler_params=pltpu.CompilerParams(
            dimension_semantics=("arbitrary",)),
        interpret=interpret,
    )(x, wg)


def _make_dispatch(T, D, E, CAP):
    info = plsc.get_sparse_core_info()
    NW = info.num_cores * info.num_subcores
    CH = 32
    t_per_w = T // NW
    n_chunks = t_per_w // CH
    DUMP = E * CAP
    mesh = plsc.VectorSubcoreMesh(core_axis_name="c", subcore_axis_name="s")

    @functools.partial(
        pl.kernel, mesh=mesh,
        out_type=jax.ShapeDtypeStruct(((E + 1) * CAP, D), jnp.float32),
        scratch_types=[
            pltpu.VMEM((2, CH, D), jnp.float32),
            pltpu.VMEM((t_per_w,), jnp.int32),
            pltpu.VMEM((t_per_w,), jnp.int32),
            pltpu.VMEM((t_per_w,), jnp.int32),
            pltpu.VMEM((t_per_w,), jnp.int32),
            pltpu.VMEM((n_chunks, CH), jnp.int32),
            pltpu.VMEM((n_chunks, CH), jnp.int32),
            pltpu.VMEM((16,), jnp.float32),
            pltpu.SemaphoreType.DMA,
            pltpu.SemaphoreType.DMA,
            pltpu.SemaphoreType.DMA,
            pltpu.SemaphoreType.DMA,
        ],
    )
    def dispatch(x_hbm, ti0_hbm, pp0_hbm, ti1_hbm, pp1_hbm, tot0_hbm,
                 disp_hbm, xb, e0b, p0b, e1b, p1b, s0b, s1b, totv,
                 seml0, seml1, sems0, sems1):
        seml = (seml0, seml1)
        sems = (sems0, sems1)
        wid = lax.axis_index("s") * info.num_cores + lax.axis_index("c")
        base = wid * t_per_w
        pltpu.sync_copy(tot0_hbm, totv)
        pltpu.sync_copy(ti0_hbm.at[pl.ds(base, t_per_w)], e0b)
        pltpu.sync_copy(pp0_hbm.at[pl.ds(base, t_per_w)], p0b)
        pltpu.sync_copy(ti1_hbm.at[pl.ds(base, t_per_w)], e1b)
        pltpu.sync_copy(pp1_hbm.at[pl.ds(base, t_per_w)], p1b)
        loads = [None] * n_chunks
        scat = [None] * n_chunks
        loads[0] = pltpu.async_copy(x_hbm.at[pl.ds(base, CH)], xb.at[0], seml[0])
        for ch in range(n_chunks):
            for h in range(CH // 16):
                sl = pl.ds(ch * CH + h * 16, 16)
                so = pl.ds(h * 16, 16)
                e0 = e0b[sl]
                p0 = p0b[sl]
                s0b[ch, so] = jnp.where(p0 < CAP, e0 * CAP + p0, DUMP)
                e1 = e1b[sl]
                p1 = p1b[sl] + _vgather(totv[...], e1).astype(jnp.int32)
                s1b[ch, so] = jnp.where(p1 < CAP, e1 * CAP + p1, DUMP)
        for ch in range(n_chunks):
            b = ch % 2
            loads[ch].wait()
            if ch >= 1:
                scat[ch - 1][0].wait()
                scat[ch - 1][1].wait()
            if ch + 1 < n_chunks:
                loads[ch + 1] = pltpu.async_copy(
                    x_hbm.at[pl.ds(base + (ch + 1) * CH, CH)],
                    xb.at[1 - b], seml[1 - b])
            scat[ch] = (
                pltpu.async_copy(xb.at[b], disp_hbm.at[s0b.at[ch]], sems[b]),
                pltpu.async_copy(xb.at[b], disp_hbm.at[s1b.at[ch]], sems[b]),
            )
        scat[n_chunks - 1][0].wait()
        scat[n_chunks - 1][1].wait()

    return dispatch


def _ffn_body(caps, disp_ref, w1_ref, w2_ref, eo_ref):
    CAP, D, FB = caps
    f = pl.program_id(1)
    dsp = disp_ref[...]
    h = jnp.dot(dsp, w1_ref[0], preferred_element_type=jnp.float32)
    h = jnp.maximum(h, 0.0)
    contrib = jnp.dot(h, w2_ref[0], preferred_element_type=jnp.float32)

    @pl.when(f == 0)
    def _first():
        eo_ref[...] = contrib

    @pl.when(f != 0)
    def _rest():
        eo_ref[...] = eo_ref[...] + contrib


def _run_ffn(disp, w1, w2, E, CAP, interpret=False):
    D = w1.shape[1]
    F = w1.shape[2]
    FB = 2048
    NF = F // FB
    return pl.pallas_call(
        functools.partial(_ffn_body, (CAP, D, FB)),
        grid=(E, NF),
        in_specs=[
            pl.BlockSpec((CAP, D), lambda e, f: (e, 0)),
            pl.BlockSpec((1, D, FB), lambda e, f: (e, 0, f)),
            pl.BlockSpec((1, FB, D), lambda e, f: (e, f, 0)),
        ],
        out_specs=pl.BlockSpec((CAP, D), lambda e, f: (e, 0)),
        out_shape=jax.ShapeDtypeStruct((E * CAP, D), jnp.float32),
        compiler_params=pltpu.CompilerParams(
            dimension_semantics=("parallel", "arbitrary")),
        interpret=interpret,
    )(disp, w1, w2)


def _make_combine(T, D, E, CAP):
    info = plsc.get_sparse_core_info()
    NW = info.num_cores * info.num_subcores
    CH = 16
    t_per_w = T // NW
    n_chunks = t_per_w // CH
    mesh = plsc.VectorSubcoreMesh(core_axis_name="c", subcore_axis_name="s")

    @functools.partial(
        pl.kernel, mesh=mesh,
        out_type=jax.ShapeDtypeStruct((T, D), jnp.float32),
        scratch_types=[
            pltpu.VMEM((4, CH, D), jnp.float32),
            pltpu.VMEM((t_per_w,), jnp.int32),
            pltpu.VMEM((t_per_w,), jnp.int32),
            pltpu.VMEM((t_per_w,), jnp.int32),
            pltpu.VMEM((t_per_w,), jnp.int32),
            pltpu.VMEM((t_per_w,), jnp.float32),
            pltpu.VMEM((t_per_w,), jnp.float32),
            pltpu.VMEM((n_chunks, CH), jnp.int32),
            pltpu.VMEM((n_chunks, CH), jnp.int32),
            pltpu.VMEM((t_per_w,), jnp.float32),
            pltpu.VMEM((t_per_w,), jnp.float32),
            pltpu.VMEM((16,), jnp.float32),
            pltpu.SemaphoreType.DMA,
            pltpu.SemaphoreType.DMA,
            pltpu.SemaphoreType.DMA,
            pltpu.SemaphoreType.DMA,
        ],
    )
    def combine(eo_hbm, ti0_hbm, pp0_hbm, ti1_hbm, pp1_hbm, wr0_hbm, wr1_hbm,
                tot0_hbm, y_hbm, g, e0b, p0b, e1b, p1b, w0r, w1r,
                r0b, r1b, w0b, w1b, totv, semg0, semg1, semw0, semw1):
        semg = (semg0, semg1)
        semw = (semw0, semw1)
        wid = lax.axis_index("s") * info.num_cores + lax.axis_index("c")
        base = wid * t_per_w
        pltpu.sync_copy(tot0_hbm, totv)
        pltpu.sync_copy(ti0_hbm.at[pl.ds(base, t_per_w)], e0b)
        pltpu.sync_copy(pp0_hbm.at[pl.ds(base, t_per_w)], p0b)
        pltpu.sync_copy(ti1_hbm.at[pl.ds(base, t_per_w)], e1b)
        pltpu.sync_copy(pp1_hbm.at[pl.ds(base, t_per_w)], p1b)
        pltpu.sync_copy(wr0_hbm.at[pl.ds(base, t_per_w)], w0r)
        pltpu.sync_copy(wr1_hbm.at[pl.ds(base, t_per_w)], w1r)
        for ch in range(n_chunks):
            sl = pl.ds(ch * CH, 16)
            so = pl.ds(0, 16)
            e0 = e0b[sl]
            p0 = p0b[sl]
            r0b[ch, so] = e0 * CAP + jnp.minimum(p0, CAP - 1)
            w0b[sl] = jnp.where(p0 < CAP, w0r[sl], 0.0)
            e1 = e1b[sl]
            p1 = p1b[sl] + _vgather(totv[...], e1).astype(jnp.int32)
            r1b[ch, so] = e1 * CAP + jnp.minimum(p1, CAP - 1)
            w1b[sl] = jnp.where(p1 < CAP, w1r[sl], 0.0)

        def gathers(ch):
            pr = ch % 2
            return (
                pltpu.async_copy(eo_hbm.at[r0b.at[ch]], g.at[2 * pr], semg[pr]),
                pltpu.async_copy(eo_hbm.at[r1b.at[ch]], g.at[2 * pr + 1], semg[pr]),
            )

        gat = [None] * n_chunks
        wrt = [None] * n_chunks
        gat[0] = gathers(0)
        for ch in range(n_chunks):
            pr = ch % 2
            gat[ch][0].wait()
            gat[ch][1].wait()
            if ch + 1 < n_chunks:
                if ch >= 1:
                    wrt[ch - 1].wait()
                gat[ch + 1] = gathers(ch + 1)
            wv0 = w0b[pl.ds(ch * CH, 16)]
            wv1 = w1b[pl.ds(ch * CH, 16)]

            def row(r, _, wv0=wv0, wv1=wv1, pr=pr):
                rv = jnp.full((16,), r, dtype=jnp.int32)
                w0v = _vgather(wv0, rv)
                w1v = _vgather(wv1, rv)
                for j in range(D // 16):
                    sl = pl.ds(j * 16, 16)
                    g[2 * pr, r, sl] = (g[2 * pr, r, sl] * w0v
                                        + g[2 * pr + 1, r, sl] * w1v)
                return 0

            lax.fori_loop(0, CH, row, 0)
            wrt[ch] = pltpu.async_copy(
                g.at[2 * pr], y_hbm.at[pl.ds(base + ch * CH, CH)], semw[pr])
        wrt[n_chunks - 2].wait()
        wrt[n_chunks - 1].wait()

    return combine


def kernel(x, wg, w1, w2):
    T, D = x.shape
    E = wg.shape[1]
    CAP = int(math.ceil(T * K / E * CAPACITY_FACTOR))

    topi, posp, topw, totals = _run_router(x, wg, E, CAP)
    ti0 = topi[:, 0]
    ti1 = topi[:, 1]
    pp0 = posp[:, 0]
    pp1 = posp[:, 1]
    wr0 = topw[:, 0]
    wr1 = topw[:, 1]
    tot0 = totals[0]

    disp = _make_dispatch(T, D, E, CAP)(x, ti0, pp0, ti1, pp1, tot0)
    eo = _run_ffn(disp, w1, w2, E, CAP)
    y = _make_combine(T, D, E, CAP)(eo, ti0, pp0, ti1, pp1, wr0, wr1, tot0)
    return y


# speedup vs baseline: 1.7306x; 1.0174x over previous
import functools
import math

import jax
import jax.numpy as jnp
from jax import lax
from jax.experimental import pallas as pl
from jax.experimental.pallas import tpu as pltpu
from jax.experimental.pallas import tpu_sc as plsc

K = 2
CAPACITY_FACTOR = 1.25


def _vgather(vec, idx):
    return lax.gather(
        vec, idx[:, None],
        lax.GatherDimensionNumbers(offset_dims=(), collapsed_slice_dims=(0,),
                                   start_index_map=(0,)),
        (1,), mode=lax.GatherScatterMode.PROMISE_IN_BOUNDS)


def _router_body(caps, x_ref, wg_ref, topi_ref, posp_ref, topw_ref, tot_ref):
    E, CAP, BT = caps
    b = pl.program_id(0)

    @pl.when(b == 0)
    def _init():
        tot_ref[...] = jnp.zeros_like(tot_ref)

    x = x_ref[...]
    wg = wg_ref[...]
    logits = jnp.dot(x, wg, preferred_element_type=jnp.float32)
    m = jnp.max(logits, axis=-1, keepdims=True)
    p = jnp.exp(logits - m)
    gates = p / jnp.sum(p, axis=-1, keepdims=True)

    lane = lax.broadcasted_iota(jnp.int32, gates.shape, 1)
    v1 = jnp.max(gates, axis=-1, keepdims=True)
    i1 = jnp.min(jnp.where(gates == v1, lane, E), axis=-1, keepdims=True)
    g2 = jnp.where(lane == i1, -jnp.inf, gates)
    v2 = jnp.max(g2, axis=-1, keepdims=True)
    i2 = jnp.min(jnp.where(g2 == v2, lane, E), axis=-1, keepdims=True)

    s = v1 + v2
    topw_ref[...] = jnp.concatenate([v1 / s, v2 / s], axis=1)
    topi_ref[...] = jnp.concatenate([i1, i2], axis=1)

    oh1 = (lane == i1).astype(jnp.float32)
    oh2 = (lane == i2).astype(jnp.float32)
    r = lax.broadcasted_iota(jnp.int32, (BT, BT), 0)
    c = lax.broadcasted_iota(jnp.int32, (BT, BT), 1)
    tri = (c < r).astype(jnp.float32)
    prev1 = jnp.dot(tri, oh1, preferred_element_type=jnp.float32)
    prev2 = jnp.dot(tri, oh2, preferred_element_type=jnp.float32)

    carry = tot_ref[...]
    pos1 = jnp.sum(oh1 * (prev1 + carry[0:1, :]), axis=-1, keepdims=True)
    pos2 = jnp.sum(oh2 * (prev2 + carry[1:2, :]), axis=-1, keepdims=True)
    posp_ref[...] = jnp.concatenate([pos1, pos2], axis=1).astype(jnp.int32)

    counts = jnp.concatenate([jnp.sum(oh1, axis=0, keepdims=True),
                              jnp.sum(oh2, axis=0, keepdims=True)], axis=0)
    tot_ref[...] = carry + counts


def _run_router(x, wg, E, CAP, interpret=False):
    T, D = x.shape
    BT = 512
    NB = T // BT
    return pl.pallas_call(
        functools.partial(_router_body, (E, CAP, BT)),
        grid=(NB,),
        in_specs=[
            pl.BlockSpec((BT, D), lambda b: (b, 0)),
            pl.BlockSpec((D, E), lambda b: (0, 0)),
        ],
        out_specs=[
            pl.BlockSpec((BT, K), lambda b: (b, 0)),
            pl.BlockSpec((BT, K), lambda b: (b, 0)),
            pl.BlockSpec((BT, K), lambda b: (b, 0)),
            pl.BlockSpec((K, E), lambda b: (0, 0)),
        ],
        out_shape=[
            jax.ShapeDtypeStruct((T, K), jnp.int32),
            jax.ShapeDtypeStruct((T, K), jnp.int32),
            jax.ShapeDtypeStruct((T, K), jnp.float32),
            jax.ShapeDtypeStruct((K, E), jnp.float32),
        ],
        compiler_params=pltpu.CompilerParams(
            dimension_semantics=("arbitrary",)),
        interpret=interpret,
    )(x, wg)


def _make_dispatch(T, D, E, CAP):
    info = plsc.get_sparse_core_info()
    NW = info.num_cores * info.num_subcores
    CH = 32
    t_per_w = T // NW
    n_chunks = t_per_w // CH
    DUMP = E * CAP
    mesh = plsc.VectorSubcoreMesh(core_axis_name="c", subcore_axis_name="s")

    @functools.partial(
        pl.kernel, mesh=mesh,
        out_type=jax.ShapeDtypeStruct(((E + 1) * CAP, D), jnp.float32),
        scratch_types=[
            pltpu.VMEM((2, CH, D), jnp.float32),
            pltpu.VMEM((t_per_w,), jnp.int32),
            pltpu.VMEM((t_per_w,), jnp.int32),
            pltpu.VMEM((t_per_w,), jnp.int32),
            pltpu.VMEM((t_per_w,), jnp.int32),
            pltpu.VMEM((n_chunks, CH), jnp.int32),
            pltpu.VMEM((n_chunks, CH), jnp.int32),
            pltpu.VMEM((16,), jnp.float32),
            pltpu.SemaphoreType.DMA,
            pltpu.SemaphoreType.DMA,
            pltpu.SemaphoreType.DMA,
            pltpu.SemaphoreType.DMA,
            pltpu.SemaphoreType.DMA,
        ],
    )
    def dispatch(x_hbm, ti0_hbm, pp0_hbm, ti1_hbm, pp1_hbm, tot0_hbm,
                 disp_hbm, xb, e0b, p0b, e1b, p1b, s0b, s1b, totv,
                 seml0, seml1, sems0, sems1, semi):
        seml = (seml0, seml1)
        sems = (sems0, sems1)
        wid = lax.axis_index("s") * info.num_cores + lax.axis_index("c")
        base = wid * t_per_w
        idx_copies = [
            pltpu.async_copy(tot0_hbm, totv, semi),
            pltpu.async_copy(ti0_hbm.at[pl.ds(base, t_per_w)], e0b, semi),
            pltpu.async_copy(pp0_hbm.at[pl.ds(base, t_per_w)], p0b, semi),
            pltpu.async_copy(ti1_hbm.at[pl.ds(base, t_per_w)], e1b, semi),
            pltpu.async_copy(pp1_hbm.at[pl.ds(base, t_per_w)], p1b, semi),
        ]
        for c in idx_copies:
            c.wait()
        loads = [None] * n_chunks
        scat = [None] * n_chunks
        loads[0] = pltpu.async_copy(x_hbm.at[pl.ds(base, CH)], xb.at[0], seml[0])
        for ch in range(n_chunks):
            for h in range(CH // 16):
                sl = pl.ds(ch * CH + h * 16, 16)
                so = pl.ds(h * 16, 16)
                e0 = e0b[sl]
                p0 = p0b[sl]
                s0b[ch, so] = jnp.where(p0 < CAP, e0 * CAP + p0, DUMP)
                e1 = e1b[sl]
                p1 = p1b[sl] + _vgather(totv[...], e1).astype(jnp.int32)
                s1b[ch, so] = jnp.where(p1 < CAP, e1 * CAP + p1, DUMP)
        for ch in range(n_chunks):
            b = ch % 2
            loads[ch].wait()
            if ch >= 1:
                scat[ch - 1][0].wait()
                scat[ch - 1][1].wait()
            if ch + 1 < n_chunks:
                loads[ch + 1] = pltpu.async_copy(
                    x_hbm.at[pl.ds(base + (ch + 1) * CH, CH)],
                    xb.at[1 - b], seml[1 - b])
            scat[ch] = (
                pltpu.async_copy(xb.at[b], disp_hbm.at[s0b.at[ch]], sems[b]),
                pltpu.async_copy(xb.at[b], disp_hbm.at[s1b.at[ch]], sems[b]),
            )
        scat[n_chunks - 1][0].wait()
        scat[n_chunks - 1][1].wait()

    return dispatch


def _ffn_body(caps, disp_ref, w1_ref, w2_ref, eo_ref):
    CAP, D, FB = caps
    f = pl.program_id(1)
    dsp = disp_ref[...]
    h = jnp.dot(dsp, w1_ref[0], preferred_element_type=jnp.float32)
    h = jnp.maximum(h, 0.0)
    contrib = jnp.dot(h, w2_ref[0], preferred_element_type=jnp.float32)

    @pl.when(f == 0)
    def _first():
        eo_ref[...] = contrib

    @pl.when(f != 0)
    def _rest():
        eo_ref[...] = eo_ref[...] + contrib


def _run_ffn(disp, w1, w2, E, CAP, interpret=False):
    D = w1.shape[1]
    F = w1.shape[2]
    FB = 2048
    NF = F // FB
    return pl.pallas_call(
        functools.partial(_ffn_body, (CAP, D, FB)),
        grid=(E, NF),
        in_specs=[
            pl.BlockSpec((CAP, D), lambda e, f: (e, 0)),
            pl.BlockSpec((1, D, FB), lambda e, f: (e, 0, f)),
            pl.BlockSpec((1, FB, D), lambda e, f: (e, f, 0)),
        ],
        out_specs=pl.BlockSpec((CAP, D), lambda e, f: (e, 0)),
        out_shape=jax.ShapeDtypeStruct((E * CAP, D), jnp.float32),
        compiler_params=pltpu.CompilerParams(
            dimension_semantics=("parallel", "arbitrary")),
        interpret=interpret,
    )(disp, w1, w2)


def _make_combine(T, D, E, CAP):
    info = plsc.get_sparse_core_info()
    NW = info.num_cores * info.num_subcores
    CH = 16
    t_per_w = T // NW
    n_chunks = t_per_w // CH
    mesh = plsc.VectorSubcoreMesh(core_axis_name="c", subcore_axis_name="s")

    @functools.partial(
        pl.kernel, mesh=mesh,
        out_type=jax.ShapeDtypeStruct((T, D), jnp.float32),
        scratch_types=[
            pltpu.VMEM((4, CH, D), jnp.float32),
            pltpu.VMEM((t_per_w,), jnp.int32),
            pltpu.VMEM((t_per_w,), jnp.int32),
            pltpu.VMEM((t_per_w,), jnp.int32),
            pltpu.VMEM((t_per_w,), jnp.int32),
            pltpu.VMEM((t_per_w,), jnp.float32),
            pltpu.VMEM((t_per_w,), jnp.float32),
            pltpu.VMEM((n_chunks, CH), jnp.int32),
            pltpu.VMEM((n_chunks, CH), jnp.int32),
            pltpu.VMEM((t_per_w,), jnp.float32),
            pltpu.VMEM((t_per_w,), jnp.float32),
            pltpu.VMEM((16,), jnp.float32),
            pltpu.SemaphoreType.DMA,
            pltpu.SemaphoreType.DMA,
            pltpu.SemaphoreType.DMA,
            pltpu.SemaphoreType.DMA,
            pltpu.SemaphoreType.DMA,
        ],
    )
    def combine(eo_hbm, ti0_hbm, pp0_hbm, ti1_hbm, pp1_hbm, wr0_hbm, wr1_hbm,
                tot0_hbm, y_hbm, g, e0b, p0b, e1b, p1b, w0r, w1r,
                r0b, r1b, w0b, w1b, totv, semg0, semg1, semw0, semw1, semi):
        semg = (semg0, semg1)
        semw = (semw0, semw1)
        wid = lax.axis_index("s") * info.num_cores + lax.axis_index("c")
        base = wid * t_per_w
        idx_copies = [
            pltpu.async_copy(tot0_hbm, totv, semi),
            pltpu.async_copy(ti0_hbm.at[pl.ds(base, t_per_w)], e0b, semi),
            pltpu.async_copy(pp0_hbm.at[pl.ds(base, t_per_w)], p0b, semi),
            pltpu.async_copy(ti1_hbm.at[pl.ds(base, t_per_w)], e1b, semi),
            pltpu.async_copy(pp1_hbm.at[pl.ds(base, t_per_w)], p1b, semi),
            pltpu.async_copy(wr0_hbm.at[pl.ds(base, t_per_w)], w0r, semi),
            pltpu.async_copy(wr1_hbm.at[pl.ds(base, t_per_w)], w1r, semi),
        ]
        for c in idx_copies:
            c.wait()
        for ch in range(n_chunks):
            sl = pl.ds(ch * CH, 16)
            so = pl.ds(0, 16)
            e0 = e0b[sl]
            p0 = p0b[sl]
            r0b[ch, so] = e0 * CAP + jnp.minimum(p0, CAP - 1)
            w0b[sl] = jnp.where(p0 < CAP, w0r[sl], 0.0)
            e1 = e1b[sl]
            p1 = p1b[sl] + _vgather(totv[...], e1).astype(jnp.int32)
            r1b[ch, so] = e1 * CAP + jnp.minimum(p1, CAP - 1)
            w1b[sl] = jnp.where(p1 < CAP, w1r[sl], 0.0)

        def gathers(ch):
            pr = ch % 2
            return (
                pltpu.async_copy(eo_hbm.at[r0b.at[ch]], g.at[2 * pr], semg[pr]),
                pltpu.async_copy(eo_hbm.at[r1b.at[ch]], g.at[2 * pr + 1], semg[pr]),
            )

        gat = [None] * n_chunks
        wrt = [None] * n_chunks
        gat[0] = gathers(0)
        for ch in range(n_chunks):
            pr = ch % 2
            gat[ch][0].wait()
            gat[ch][1].wait()
            if ch + 1 < n_chunks:
                if ch >= 1:
                    wrt[ch - 1].wait()
                gat[ch + 1] = gathers(ch + 1)
            wv0 = w0b[pl.ds(ch * CH, 16)]
            wv1 = w1b[pl.ds(ch * CH, 16)]

            def row(r, _, wv0=wv0, wv1=wv1, pr=pr):
                rv = jnp.full((16,), r, dtype=jnp.int32)
                w0v = _vgather(wv0, rv)
                w1v = _vgather(wv1, rv)
                for j in range(D // 16):
                    sl = pl.ds(j * 16, 16)
                    g[2 * pr, r, sl] = (g[2 * pr, r, sl] * w0v
                                        + g[2 * pr + 1, r, sl] * w1v)
                return 0

            lax.fori_loop(0, CH, row, 0)
            wrt[ch] = pltpu.async_copy(
                g.at[2 * pr], y_hbm.at[pl.ds(base + ch * CH, CH)], semw[pr])
        wrt[n_chunks - 2].wait()
        wrt[n_chunks - 1].wait()

    return combine


def kernel(x, wg, w1, w2):
    T, D = x.shape
    E = wg.shape[1]
    CAP = int(math.ceil(T * K / E * CAPACITY_FACTOR))

    topi, posp, topw, totals = _run_router(x, wg, E, CAP)
    ti0 = topi[:, 0]
    ti1 = topi[:, 1]
    pp0 = posp[:, 0]
    pp1 = posp[:, 1]
    wr0 = topw[:, 0]
    wr1 = topw[:, 1]
    tot0 = totals[0]

    disp = _make_dispatch(T, D, E, CAP)(x, ti0, pp0, ti1, pp1, tot0)
    eo = _run_ffn(disp, w1, w2, E, CAP)
    y = _make_combine(T, D, E, CAP)(eo, ti0, pp0, ti1, pp1, wr0, wr1, tot0)
    return y


# speedup vs baseline: 1.7410x; 1.0060x over previous
import functools
import math

import jax
import jax.numpy as jnp
from jax import lax
from jax.experimental import pallas as pl
from jax.experimental.pallas import tpu as pltpu
from jax.experimental.pallas import tpu_sc as plsc

K = 2
CAPACITY_FACTOR = 1.25


def _vgather(vec, idx):
    return lax.gather(
        vec, idx[:, None],
        lax.GatherDimensionNumbers(offset_dims=(), collapsed_slice_dims=(0,),
                                   start_index_map=(0,)),
        (1,), mode=lax.GatherScatterMode.PROMISE_IN_BOUNDS)


def _router_body(caps, x_ref, wg_ref, topi_ref, posp_ref, topw_ref, tot_ref):
    E, CAP, BT = caps
    b = pl.program_id(0)

    @pl.when(b == 0)
    def _init():
        tot_ref[...] = jnp.zeros_like(tot_ref)

    x = x_ref[...]
    wg = wg_ref[...]
    logits = jnp.dot(x, wg, preferred_element_type=jnp.float32)
    m = jnp.max(logits, axis=-1, keepdims=True)
    p = jnp.exp(logits - m)
    gates = p / jnp.sum(p, axis=-1, keepdims=True)

    lane = lax.broadcasted_iota(jnp.int32, gates.shape, 1)
    v1 = jnp.max(gates, axis=-1, keepdims=True)
    i1 = jnp.min(jnp.where(gates == v1, lane, E), axis=-1, keepdims=True)
    g2 = jnp.where(lane == i1, -jnp.inf, gates)
    v2 = jnp.max(g2, axis=-1, keepdims=True)
    i2 = jnp.min(jnp.where(g2 == v2, lane, E), axis=-1, keepdims=True)

    s = v1 + v2
    topw_ref[...] = jnp.concatenate([v1 / s, v2 / s], axis=1)
    topi_ref[...] = jnp.concatenate([i1, i2], axis=1)

    oh1 = (lane == i1).astype(jnp.float32)
    oh2 = (lane == i2).astype(jnp.float32)
    r = lax.broadcasted_iota(jnp.int32, (BT, BT), 0)
    c = lax.broadcasted_iota(jnp.int32, (BT, BT), 1)
    tri = (c < r).astype(jnp.float32)
    prev1 = jnp.dot(tri, oh1, preferred_element_type=jnp.float32)
    prev2 = jnp.dot(tri, oh2, preferred_element_type=jnp.float32)

    carry = tot_ref[...]
    pos1 = jnp.sum(oh1 * (prev1 + carry[0:1, :]), axis=-1, keepdims=True)
    pos2 = jnp.sum(oh2 * (prev2 + carry[1:2, :]), axis=-1, keepdims=True)
    posp_ref[...] = jnp.concatenate([pos1, pos2], axis=1).astype(jnp.int32)

    counts = jnp.concatenate([jnp.sum(oh1, axis=0, keepdims=True),
                              jnp.sum(oh2, axis=0, keepdims=True)], axis=0)
    tot_ref[...] = carry + counts


def _run_router(x, wg, E, CAP, interpret=False):
    T, D = x.shape
    BT = 1024
    NB = T // BT
    return pl.pallas_call(
        functools.partial(_router_body, (E, CAP, BT)),
        grid=(NB,),
        in_specs=[
            pl.BlockSpec((BT, D), lambda b: (b, 0)),
            pl.BlockSpec((D, E), lambda b: (0, 0)),
        ],
        out_specs=[
            pl.BlockSpec((BT, K), lambda b: (b, 0)),
            pl.BlockSpec((BT, K), lambda b: (b, 0)),
            pl.BlockSpec((BT, K), lambda b: (b, 0)),
            pl.BlockSpec((K, E), lambda b: (0, 0)),
        ],
        out_shape=[
            jax.ShapeDtypeStruct((T, K), jnp.int32),
            jax.ShapeDtypeStruct((T, K), jnp.int32),
            jax.ShapeDtypeStruct((T, K), jnp.float32),
            jax.ShapeDtypeStruct((K, E), jnp.float32),
        ],
        compiler_params=pltpu.CompilerParams(
            dimension_semantics=("arbitrary",)),
        interpret=interpret,
    )(x, wg)


def _make_dispatch(T, D, E, CAP):
    info = plsc.get_sparse_core_info()
    NW = info.num_cores * info.num_subcores
    CH = 32
    t_per_w = T // NW
    n_chunks = t_per_w // CH
    DUMP = E * CAP
    mesh = plsc.VectorSubcoreMesh(core_axis_name="c", subcore_axis_name="s")

    @functools.partial(
        pl.kernel, mesh=mesh,
        out_type=jax.ShapeDtypeStruct(((E + 1) * CAP, D), jnp.float32),
        scratch_types=[
            pltpu.VMEM((2, CH, D), jnp.float32),
            pltpu.VMEM((t_per_w,), jnp.int32),
            pltpu.VMEM((t_per_w,), jnp.int32),
            pltpu.VMEM((t_per_w,), jnp.int32),
            pltpu.VMEM((t_per_w,), jnp.int32),
            pltpu.VMEM((n_chunks, CH), jnp.int32),
            pltpu.VMEM((n_chunks, CH), jnp.int32),
            pltpu.VMEM((16,), jnp.float32),
            pltpu.SemaphoreType.DMA,
            pltpu.SemaphoreType.DMA,
            pltpu.SemaphoreType.DMA,
            pltpu.SemaphoreType.DMA,
            pltpu.SemaphoreType.DMA,
        ],
    )
    def dispatch(x_hbm, ti0_hbm, pp0_hbm, ti1_hbm, pp1_hbm, tot0_hbm,
                 disp_hbm, xb, e0b, p0b, e1b, p1b, s0b, s1b, totv,
                 seml0, seml1, sems0, sems1, semi):
        seml = (seml0, seml1)
        sems = (sems0, sems1)
        wid = lax.axis_index("s") * info.num_cores + lax.axis_index("c")
        base = wid * t_per_w
        idx_copies = [
            pltpu.async_copy(tot0_hbm, totv, semi),
            pltpu.async_copy(ti0_hbm.at[pl.ds(base, t_per_w)], e0b, semi),
            pltpu.async_copy(pp0_hbm.at[pl.ds(base, t_per_w)], p0b, semi),
            pltpu.async_copy(ti1_hbm.at[pl.ds(base, t_per_w)], e1b, semi),
            pltpu.async_copy(pp1_hbm.at[pl.ds(base, t_per_w)], p1b, semi),
        ]
        for c in idx_copies:
            c.wait()
        loads = [None] * n_chunks
        scat = [None] * n_chunks
        loads[0] = pltpu.async_copy(x_hbm.at[pl.ds(base, CH)], xb.at[0], seml[0])
        for ch in range(n_chunks):
            for h in range(CH // 16):
                sl = pl.ds(ch * CH + h * 16, 16)
                so = pl.ds(h * 16, 16)
                e0 = e0b[sl]
                p0 = p0b[sl]
                s0b[ch, so] = jnp.where(p0 < CAP, e0 * CAP + p0, DUMP)
                e1 = e1b[sl]
                p1 = p1b[sl] + _vgather(totv[...], e1).astype(jnp.int32)
                s1b[ch, so] = jnp.where(p1 < CAP, e1 * CAP + p1, DUMP)
        for ch in range(n_chunks):
            b = ch % 2
            loads[ch].wait()
            if ch >= 1:
                scat[ch - 1][0].wait()
                scat[ch - 1][1].wait()
            if ch + 1 < n_chunks:
                loads[ch + 1] = pltpu.async_copy(
                    x_hbm.at[pl.ds(base + (ch + 1) * CH, CH)],
                    xb.at[1 - b], seml[1 - b])
            scat[ch] = (
                pltpu.async_copy(xb.at[b], disp_hbm.at[s0b.at[ch]], sems[b]),
                pltpu.async_copy(xb.at[b], disp_hbm.at[s1b.at[ch]], sems[b]),
            )
        scat[n_chunks - 1][0].wait()
        scat[n_chunks - 1][1].wait()

    return dispatch


def _ffn_body(caps, disp_ref, w1_ref, w2_ref, eo_ref):
    CAP, D, FB = caps
    f = pl.program_id(1)
    dsp = disp_ref[...]
    h = jnp.dot(dsp, w1_ref[0], preferred_element_type=jnp.float32)
    h = jnp.maximum(h, 0.0)
    contrib = jnp.dot(h, w2_ref[0], preferred_element_type=jnp.float32)

    @pl.when(f == 0)
    def _first():
        eo_ref[...] = contrib

    @pl.when(f != 0)
    def _rest():
        eo_ref[...] = eo_ref[...] + contrib


def _run_ffn(disp, w1, w2, E, CAP, interpret=False):
    D = w1.shape[1]
    F = w1.shape[2]
    FB = 2048
    NF = F // FB
    return pl.pallas_call(
        functools.partial(_ffn_body, (CAP, D, FB)),
        grid=(E, NF),
        in_specs=[
            pl.BlockSpec((CAP, D), lambda e, f: (e, 0)),
            pl.BlockSpec((1, D, FB), lambda e, f: (e, 0, f)),
            pl.BlockSpec((1, FB, D), lambda e, f: (e, f, 0)),
        ],
        out_specs=pl.BlockSpec((CAP, D), lambda e, f: (e, 0)),
        out_shape=jax.ShapeDtypeStruct((E * CAP, D), jnp.float32),
        compiler_params=pltpu.CompilerParams(
            dimension_semantics=("parallel", "arbitrary")),
        interpret=interpret,
    )(disp, w1, w2)


def _make_combine(T, D, E, CAP):
    info = plsc.get_sparse_core_info()
    NW = info.num_cores * info.num_subcores
    CH = 16
    t_per_w = T // NW
    n_chunks = t_per_w // CH
    mesh = plsc.VectorSubcoreMesh(core_axis_name="c", subcore_axis_name="s")

    @functools.partial(
        pl.kernel, mesh=mesh,
        out_type=jax.ShapeDtypeStruct((T, D), jnp.float32),
        scratch_types=[
            pltpu.VMEM((4, CH, D), jnp.float32),
            pltpu.VMEM((t_per_w,), jnp.int32),
            pltpu.VMEM((t_per_w,), jnp.int32),
            pltpu.VMEM((t_per_w,), jnp.int32),
            pltpu.VMEM((t_per_w,), jnp.int32),
            pltpu.VMEM((t_per_w,), jnp.float32),
            pltpu.VMEM((t_per_w,), jnp.float32),
            pltpu.VMEM((n_chunks, CH), jnp.int32),
            pltpu.VMEM((n_chunks, CH), jnp.int32),
            pltpu.VMEM((t_per_w,), jnp.float32),
            pltpu.VMEM((t_per_w,), jnp.float32),
            pltpu.VMEM((16,), jnp.float32),
            pltpu.SemaphoreType.DMA,
            pltpu.SemaphoreType.DMA,
            pltpu.SemaphoreType.DMA,
            pltpu.SemaphoreType.DMA,
            pltpu.SemaphoreType.DMA,
        ],
    )
    def combine(eo_hbm, ti0_hbm, pp0_hbm, ti1_hbm, pp1_hbm, wr0_hbm, wr1_hbm,
                tot0_hbm, y_hbm, g, e0b, p0b, e1b, p1b, w0r, w1r,
                r0b, r1b, w0b, w1b, totv, semg0, semg1, semw0, semw1, semi):
        semg = (semg0, semg1)
        semw = (semw0, semw1)
        wid = lax.axis_index("s") * info.num_cores + lax.axis_index("c")
        base = wid * t_per_w
        idx_copies = [
            pltpu.async_copy(tot0_hbm, totv, semi),
            pltpu.async_copy(ti0_hbm.at[pl.ds(base, t_per_w)], e0b, semi),
            pltpu.async_copy(pp0_hbm.at[pl.ds(base, t_per_w)], p0b, semi),
            pltpu.async_copy(ti1_hbm.at[pl.ds(base, t_per_w)], e1b, semi),
            pltpu.async_copy(pp1_hbm.at[pl.ds(base, t_per_w)], p1b, semi),
            pltpu.async_copy(wr0_hbm.at[pl.ds(base, t_per_w)], w0r, semi),
            pltpu.async_copy(wr1_hbm.at[pl.ds(base, t_per_w)], w1r, semi),
        ]
        for c in idx_copies:
            c.wait()
        for ch in range(n_chunks):
            sl = pl.ds(ch * CH, 16)
            so = pl.ds(0, 16)
            e0 = e0b[sl]
            p0 = p0b[sl]
            r0b[ch, so] = e0 * CAP + jnp.minimum(p0, CAP - 1)
            w0b[sl] = jnp.where(p0 < CAP, w0r[sl], 0.0)
            e1 = e1b[sl]
            p1 = p1b[sl] + _vgather(totv[...], e1).astype(jnp.int32)
            r1b[ch, so] = e1 * CAP + jnp.minimum(p1, CAP - 1)
            w1b[sl] = jnp.where(p1 < CAP, w1r[sl], 0.0)

        def gathers(ch):
            pr = ch % 2
            return (
                pltpu.async_copy(eo_hbm.at[r0b.at[ch]], g.at[2 * pr], semg[pr]),
                pltpu.async_copy(eo_hbm.at[r1b.at[ch]], g.at[2 * pr + 1], semg[pr]),
            )

        gat = [None] * n_chunks
        wrt = [None] * n_chunks
        gat[0] = gathers(0)
        for ch in range(n_chunks):
            pr = ch % 2
            gat[ch][0].wait()
            gat[ch][1].wait()
            if ch + 1 < n_chunks:
                if ch >= 1:
                    wrt[ch - 1].wait()
                gat[ch + 1] = gathers(ch + 1)
            wv0 = w0b[pl.ds(ch * CH, 16)]
            wv1 = w1b[pl.ds(ch * CH, 16)]

            def row(r, _, wv0=wv0, wv1=wv1, pr=pr):
                rv = jnp.full((16,), r, dtype=jnp.int32)
                w0v = _vgather(wv0, rv)
                w1v = _vgather(wv1, rv)
                for j in range(D // 16):
                    sl = pl.ds(j * 16, 16)
                    g[2 * pr, r, sl] = (g[2 * pr, r, sl] * w0v
                                        + g[2 * pr + 1, r, sl] * w1v)
                return 0

            lax.fori_loop(0, CH, row, 0)
            wrt[ch] = pltpu.async_copy(
                g.at[2 * pr], y_hbm.at[pl.ds(base + ch * CH, CH)], semw[pr])
        wrt[n_chunks - 2].wait()
        wrt[n_chunks - 1].wait()

    return combine


def kernel(x, wg, w1, w2):
    T, D = x.shape
    E = wg.shape[1]
    CAP = int(math.ceil(T * K / E * CAPACITY_FACTOR))

    topi, posp, topw, totals = _run_router(x, wg, E, CAP)
    ti0 = topi[:, 0]
    ti1 = topi[:, 1]
    pp0 = posp[:, 0]
    pp1 = posp[:, 1]
    wr0 = topw[:, 0]
    wr1 = topw[:, 1]
    tot0 = totals[0]

    disp = _make_dispatch(T, D, E, CAP)(x, ti0, pp0, ti1, pp1, tot0)
    eo = _run_ffn(disp, w1, w2, E, CAP)
    y = _make_combine(T, D, E, CAP)(eo, ti0, pp0, ti1, pp1, wr0, wr1, tot0)
    return y
